# Initial kernel scaffold; baseline (speedup 1.0000x reference)
import jax
import jax.numpy as jnp
from jax.experimental import pallas as pl


def kernel(x, token_table, pos_table):
    raise NotImplementedError("write your pallas kernel here")


# speedup vs baseline: 1.3956x; 1.3956x over previous
import functools

import jax
import jax.numpy as jnp
from jax import lax
from jax.experimental import pallas as pl
from jax.experimental.pallas import tpu as pltpu
from jax.experimental.pallas import tpu_sc as plsc

MAXLEN = 50
DIM = 64
BATCH = 16384

NC = 2
NS = 16
NW = NC * NS

N = BATCH * MAXLEN
CHUNK = 2 * MAXLEN
NCHUNK = N // CHUNK
CPW = NCHUNK // NW
LANES = 16
VPR = DIM // LANES


def _body(x2, table, pos, out, idx_v, rows_v, pos_v, sem):
    wid = lax.axis_index("s") * NC + lax.axis_index("c")
    base = wid * CPW

    pltpu.sync_copy(x2.at[pl.ds(base, CPW)], idx_v)
    pltpu.sync_copy(pos, pos_v)

    def chunk_body(g, carry):
        cid = base + g
        pltpu.async_copy(table.at[idx_v.at[g]], rows_v, sem).wait()
        for r in range(CHUNK):
            pr = r % MAXLEN
            for c in range(VPR):
                sl = pl.ds(c * LANES, LANES)
                rows_v[r, sl] = rows_v[r, sl] + pos_v[pr, sl]
        pltpu.sync_copy(rows_v, out.at[cid])
        return carry

    lax.fori_loop(0, CPW, chunk_body, 0)


@functools.partial(
    pl.kernel,
    mesh=plsc.VectorSubcoreMesh(core_axis_name="c", subcore_axis_name="s"),
    out_type=jax.ShapeDtypeStruct((NCHUNK, CHUNK, DIM), jnp.float32),
    scratch_types=[
        pltpu.VMEM((CPW, CHUNK), jnp.int32),
        pltpu.VMEM((CHUNK, DIM), jnp.float32),
        pltpu.VMEM((DIM, DIM), jnp.float32),
        pltpu.SemaphoreType.DMA,
    ],
    compiler_params=pltpu.CompilerParams(use_tc_tiling_on_sc=False),
)
def _sc_kernel(x2, table, pos, out, idx_v, rows_v, pos_v, sem):
    _body(x2, table, pos, out, idx_v, rows_v, pos_v, sem)


def kernel(x, token_table, pos_table):
    x2 = x.astype(jnp.int32).reshape(NCHUNK, CHUNK)
    out = _sc_kernel(x2, token_table, pos_table)
    return out.reshape(BATCH, MAXLEN, DIM)


# speedup vs baseline: 1.5144x; 1.0851x over previous
import functools

import jax
import jax.numpy as jnp
from jax import lax
from jax.experimental import pallas as pl
from jax.experimental.pallas import tpu as pltpu
from jax.experimental.pallas import tpu_sc as plsc

MAXLEN = 50
DIM = 64
BATCH = 16384

NC = 2
NS = 16
NW = NC * NS

N = BATCH * MAXLEN
CHUNK = 2 * MAXLEN
NCHUNK = N // CHUNK
CPW = NCHUNK // NW
LANES = 16
VPR = DIM // LANES
NBUF = 4


def _body(x2, table, pos, out, idx_v, rows, pos_v, gsems, ssems):
    wid = lax.axis_index("s") * NC + lax.axis_index("c")
    base = wid * CPW

    pltpu.sync_copy(x2.at[pl.ds(base, CPW)], idx_v)
    pltpu.sync_copy(pos, pos_v)

    def start_gather(g, b):
        pltpu.async_copy(table.at[idx_v.at[g]], rows[b], gsems[b])

    def wait_gather(b):
        pltpu.make_async_copy(table.at[idx_v.at[0]], rows[b], gsems[b]).wait()

    def start_store(g, b):
        pltpu.async_copy(rows[b], out.at[base + g], ssems[b])

    def wait_store(b):
        pltpu.make_async_copy(rows[b], out.at[base], ssems[b]).wait()

    def add_pos(b):
        for pr in range(MAXLEN):
            for c in range(VPR):
                sl = pl.ds(c * LANES, LANES)
                p = pos_v[pr, sl]
                plsc.addupdate(rows[b].at[pr, sl], p)
                plsc.addupdate(rows[b].at[pr + MAXLEN, sl], p)

    start_gather(0, 0)
    start_gather(1, 1)

    def block_body(i, carry):
        for k in range(NBUF):
            g = i * NBUF + k
            b = k
            nb = (k + 2) % NBUF

            @pl.when(g >= 2)
            def _():
                wait_store(nb)

            @pl.when(g < CPW - 2)
            def _():
                start_gather(g + 2, nb)

            wait_gather(b)
            add_pos(b)
            start_store(g, b)
        return carry

    lax.fori_loop(0, CPW // NBUF, block_body, 0)

    wait_store((CPW - 2) % NBUF)
    wait_store((CPW - 1) % NBUF)


@functools.partial(
    pl.kernel,
    mesh=plsc.VectorSubcoreMesh(core_axis_name="c", subcore_axis_name="s"),
    out_type=jax.ShapeDtypeStruct((NCHUNK, CHUNK, DIM), jnp.float32),
    scratch_types=[
        pltpu.VMEM((CPW, CHUNK), jnp.int32),
        [pltpu.VMEM((CHUNK, DIM), jnp.float32) for _ in range(NBUF)],
        pltpu.VMEM((DIM, DIM), jnp.float32),
        [pltpu.SemaphoreType.DMA for _ in range(NBUF)],
        [pltpu.SemaphoreType.DMA for _ in range(NBUF)],
    ],
    compiler_params=pltpu.CompilerParams(use_tc_tiling_on_sc=False),
)
def _sc_kernel(x2, table, pos, out, idx_v, rows, pos_v, gsems, ssems):
    _body(x2, table, pos, out, idx_v, rows, pos_v, gsems, ssems)


def kernel(x, token_table, pos_table):
    x2 = x.astype(jnp.int32).reshape(NCHUNK, CHUNK)
    out = _sc_kernel(x2, token_table, pos_table)
    return out.reshape(BATCH, MAXLEN, DIM)
